# baseline (speedup 1.0000x reference)
import jax
import jax.numpy as jnp
from jax import lax
from jax.experimental import pallas as pl
from jax.experimental.pallas import tpu as pltpu
from jax.experimental.pallas import tpu_sc as plsc

N = 10000
E = 320000
D = 128
H = 128
G = 64
OUT = 6

NC = 2
NS = 16
NW = NC * NS

NP = 10240
ROWS_PER_TILE = NP // NS
CHUNK = 128
K = 80
TOTC = NW * K

SG = 40
EPAD = TOTC * CHUNK

RB = 1024
GRID = NP // RB


DEGL = 128


def _deg_body(dst_hbm, out_hbm, dst_v, buf, acc, sem):
    c = lax.axis_index("c")
    s = lax.axis_index("s")
    wid = s * NC + c

    zv = jnp.zeros((16,), jnp.float32)

    @pl.loop(0, CHUNK)
    def _fill_zero(i):
        @pl.loop(0, DEGL // 16)
        def _fz(k):
            buf[i, pl.ds(k * 16, 16)] = zv

    @pl.loop(0, ROWS_PER_TILE // CHUNK)
    def _zero_stripe(z):
        pltpu.sync_copy(
            buf, acc.at[pl.ds(s * ROWS_PER_TILE + z * CHUNK, CHUNK)])

    @pl.loop(0, CHUNK)
    def _fill_ones(i):
        @pl.loop(0, DEGL // 16)
        def _fo(k):
            buf[i, pl.ds(k * 16, 16)] = zv + 1.0

    pltpu.sync_copy(dst_hbm.at[pl.ds(wid * K, K)], dst_v)
    plsc.subcore_barrier()

    @pl.loop(0, K)
    def _scatter(j):
        pltpu.sync_copy(buf, acc.at[dst_v.at[j]], add=True)

    plsc.subcore_barrier()
    pltpu.sync_copy(
        acc.at[pl.ds(s * ROWS_PER_TILE, ROWS_PER_TILE)],
        out_hbm.at[c].at[pl.ds(s * ROWS_PER_TILE, ROWS_PER_TILE)],
    )


def _make_deg_kernel():
    mesh = plsc.VectorSubcoreMesh(core_axis_name="c", subcore_axis_name="s")
    return pl.kernel(
        _deg_body,
        out_type=jax.ShapeDtypeStruct((NC, NP, DEGL), jnp.float32),
        mesh=mesh,
        scratch_types=[
            pltpu.VMEM((K, CHUNK), jnp.int32),
            pltpu.VMEM((CHUNK, DEGL), jnp.float32),
            pltpu.VMEM_SHARED((NP, DEGL), jnp.float32),
            pltpu.SemaphoreType.DMA,
        ],
    )


def _agg_body(g_hbm, src_hbm, dst_hbm, out_hbm, src_v, dst_v, rows,
              acc, sem0, sem1):
    c = lax.axis_index("c")
    s = lax.axis_index("s")

    zv = jnp.zeros((16,), jnp.float32)

    @pl.loop(0, CHUNK)
    def _fill_zero(i):
        @pl.loop(0, D // 16)
        def _fz(k):
            rows[0, i, pl.ds(k * 16, 16)] = zv

    @pl.loop(0, ROWS_PER_TILE // CHUNK)
    def _zero_stripe(z):
        pltpu.sync_copy(
            rows.at[0],
            acc.at[pl.ds(s * ROWS_PER_TILE + z * CHUNK, CHUNK)],
        )

    plsc.subcore_barrier()

    def _stage(stage_base):
        pltpu.sync_copy(src_hbm.at[pl.ds(stage_base, SG)], src_v)
        pltpu.sync_copy(dst_hbm.at[pl.ds(stage_base, SG)], dst_v)

        pltpu.async_copy(g_hbm.at[src_v.at[0]], rows.at[0], sem0)
        pltpu.async_copy(g_hbm.at[src_v.at[1]], rows.at[1], sem1)

        @pl.loop(0, SG, step=2)
        def _main(j):
            for b in range(2):
                jj = j + b
                sem = sem0 if b == 0 else sem1
                pltpu.make_async_copy(g_hbm.at[src_v.at[jj]], rows.at[b],
                                      sem).wait()
                pltpu.sync_copy(rows.at[b], acc.at[dst_v.at[jj]], add=True)

                @pl.when(jj + 2 < SG)
                def _next():
                    pltpu.async_copy(g_hbm.at[src_v.at[jj + 2]],
                                     rows.at[b], sem)

    wid = s * NC + c
    for st in range(K // SG):
        _stage(wid * K + st * SG)

    plsc.subcore_barrier()
    pltpu.sync_copy(
        acc.at[pl.ds(s * ROWS_PER_TILE, ROWS_PER_TILE)],
        out_hbm.at[c].at[pl.ds(s * ROWS_PER_TILE, ROWS_PER_TILE)],
    )


def _make_agg_kernel():
    mesh = plsc.VectorSubcoreMesh(core_axis_name="c", subcore_axis_name="s")
    return pl.kernel(
        _agg_body,
        out_type=jax.ShapeDtypeStruct((NC, NP, D), jnp.float32),
        mesh=mesh,
        scratch_types=[
            pltpu.VMEM((SG, CHUNK), jnp.int32),
            pltpu.VMEM((SG, CHUNK), jnp.int32),
            pltpu.VMEM((2, CHUNK, D), jnp.float32),
            pltpu.VMEM_SHARED((NP, D), jnp.float32),
            pltpu.SemaphoreType.DMA,
            pltpu.SemaphoreType.DMA,
        ],
    )


def _tc_mm_body(x_ref, w_ref, h_ref):
    h_ref[...] = jnp.dot(x_ref[...], w_ref[...],
                         preferred_element_type=jnp.float32)


def _tc_mm(x_p, W):
    return pl.pallas_call(
        _tc_mm_body,
        grid=(GRID,),
        in_specs=[
            pl.BlockSpec((RB, D), lambda i: (i, 0)),
            pl.BlockSpec((D, H), lambda i: (0, 0)),
        ],
        out_specs=pl.BlockSpec((RB, H), lambda i: (i, 0)),
        out_shape=jax.ShapeDtypeStruct((NP, H), jnp.float32),
    )(x_p, W)


def _tc_a_body(degp_ref, h_ref, g1_ref, dm_ref):
    i = pl.program_id(0)
    d = degp_ref[0, :, 0:1] + degp_ref[1, :, 0:1] + 1.0
    d = jnp.broadcast_to(d, (RB, H))
    rows = i * RB + lax.broadcasted_iota(jnp.int32, (RB, H), 0)
    dm = jnp.where(rows < N, lax.rsqrt(d), 0.0)
    g1_ref[...] = dm * h_ref[...]
    dm_ref[...] = dm


def _tc_a(degp, h1raw):
    return pl.pallas_call(
        _tc_a_body,
        grid=(GRID,),
        in_specs=[
            pl.BlockSpec((NC, RB, DEGL), lambda i: (0, i, 0)),
            pl.BlockSpec((RB, H), lambda i: (i, 0)),
        ],
        out_specs=[
            pl.BlockSpec((RB, H), lambda i: (i, 0)),
            pl.BlockSpec((RB, H), lambda i: (i, 0)),
        ],
        out_shape=[
            jax.ShapeDtypeStruct((NP, H), jnp.float32),
            jax.ShapeDtypeStruct((NP, H), jnp.float32),
        ],
    )(degp, h1raw)


def _tc_b_body(aggp_ref, g1_ref, dm_ref, w2_ref, b1_ref, g2_ref):
    dm = dm_ref[...]
    agg = aggp_ref[0] + aggp_ref[1] + g1_ref[...]
    h1 = jnp.maximum(dm * agg + b1_ref[...], 0.0)
    h2 = jnp.dot(h1, w2_ref[...], preferred_element_type=jnp.float32)
    g2_ref[...] = dm * h2


def _tc_b(aggp, g1, dm, W2, b1_2d):
    return pl.pallas_call(
        _tc_b_body,
        grid=(GRID,),
        in_specs=[
            pl.BlockSpec((NC, RB, H), lambda i: (0, i, 0)),
            pl.BlockSpec((RB, H), lambda i: (i, 0)),
            pl.BlockSpec((RB, H), lambda i: (i, 0)),
            pl.BlockSpec((H, H), lambda i: (0, 0)),
            pl.BlockSpec((1, H), lambda i: (0, 0)),
        ],
        out_specs=pl.BlockSpec((RB, H), lambda i: (i, 0)),
        out_shape=jax.ShapeDtypeStruct((NP, H), jnp.float32),
    )(aggp, g1, dm, W2, b1_2d)


def _tc_c_body(aggp_ref, g2_ref, dm_ref, b2_ref, batch_ref, linw_ref,
               linb_ref, logits_ref, emb_ref, sums, counts):
    i = pl.program_id(0)
    dm = dm_ref[...]
    agg = aggp_ref[0] + aggp_ref[1] + g2_ref[...]
    h2 = jnp.maximum(dm * agg + b2_ref[...], 0.0)

    bvec = batch_ref[0]
    oh_t = (jnp.broadcast_to(bvec, (G, RB))
            == lax.broadcasted_iota(jnp.int32, (G, RB), 0)).astype(jnp.float32)

    @pl.when(i == 0)
    def _init():
        sums[...] = jnp.zeros((G, H), jnp.float32)
        counts[...] = jnp.zeros((G, 1), jnp.float32)

    sums[...] += jnp.dot(oh_t, h2, preferred_element_type=jnp.float32)
    counts[...] += jnp.sum(oh_t, axis=1, keepdims=True)

    @pl.when(i == GRID - 1)
    def _fin():
        cnt = jnp.broadcast_to(jnp.maximum(counts[...], 1.0), (G, H))
        emb = sums[...] / cnt
        emb_ref[...] = emb
        logits_ref[...] = (
            jnp.dot(emb, linw_ref[...], preferred_element_type=jnp.float32)
            + linb_ref[...]
        )


def _tc_c(aggp, g2, dm, b2_2d, batch3d, lin_W, lin_b_2d):
    return pl.pallas_call(
        _tc_c_body,
        grid=(GRID,),
        in_specs=[
            pl.BlockSpec((NC, RB, H), lambda i: (0, i, 0)),
            pl.BlockSpec((RB, H), lambda i: (i, 0)),
            pl.BlockSpec((RB, H), lambda i: (i, 0)),
            pl.BlockSpec((1, H), lambda i: (0, 0)),
            pl.BlockSpec((1, 1, RB), lambda i: (i, 0, 0)),
            pl.BlockSpec((H, OUT), lambda i: (0, 0)),
            pl.BlockSpec((1, OUT), lambda i: (0, 0)),
        ],
        out_specs=[
            pl.BlockSpec((G, OUT), lambda i: (0, 0)),
            pl.BlockSpec((G, H), lambda i: (0, 0)),
        ],
        out_shape=[
            jax.ShapeDtypeStruct((G, OUT), jnp.float32),
            jax.ShapeDtypeStruct((G, H), jnp.float32),
        ],
        scratch_shapes=[
            pltpu.VMEM((G, H), jnp.float32),
            pltpu.VMEM((G, 1), jnp.float32),
        ],
    )(aggp, g2, dm, b2_2d, batch3d, lin_W, lin_b_2d)


@jax.jit
def kernel(x, edge_index, batch, W1, b1, W2, b2, lin_W, lin_b):
    src = edge_index[0]
    dst = edge_index[1]
    pad_i = jnp.arange(EPAD - E, dtype=jnp.int32)
    srcs = jnp.concatenate([src, pad_i % N]).reshape(TOTC, CHUNK)
    dsts = jnp.concatenate([dst, N + pad_i % (NP - N)]).reshape(TOTC, CHUNK)

    x_p = jnp.pad(x, ((0, NP - N), (0, 0)))
    batch3d = jnp.pad(batch, (0, NP - N), constant_values=G).reshape(
        GRID, 1, RB)

    deg_k = _make_deg_kernel()
    agg_k = _make_agg_kernel()

    degp = deg_k(dsts)
    h1raw = _tc_mm(x_p, W1)
    g1, dm = _tc_a(degp, h1raw)
    agg1 = agg_k(g1, srcs, dsts)
    g2 = _tc_b(agg1, g1, dm, W2, b1.reshape(1, H))
    agg2 = agg_k(g2, srcs, dsts)
    logits, embedding = _tc_c(agg2, g2, dm, b2.reshape(1, H), batch3d,
                              lin_W, lin_b.reshape(1, OUT))
    return (logits, embedding)


# speedup vs baseline: 1.0401x; 1.0401x over previous
import jax
import jax.numpy as jnp
from jax import lax
from jax.experimental import pallas as pl
from jax.experimental.pallas import tpu as pltpu
from jax.experimental.pallas import tpu_sc as plsc

N = 10000
E = 320000
D = 128
H = 128
G = 64
OUT = 6

NC = 2
NS = 16
NW = NC * NS

NP = 10240
ROWS_PER_TILE = NP // NS
CHUNK = 128
K = 80
TOTC = NW * K

SG = 40
EPAD = TOTC * CHUNK

ACH = 64
AK = 160
ATOTC = NW * AK
ASG = 40
NBUF = 4

RB = 1024
GRID = NP // RB


DEGL = 128


def _deg_body(dst_hbm, out_hbm, dst_v, buf, acc, sem):
    c = lax.axis_index("c")
    s = lax.axis_index("s")
    wid = s * NC + c

    zv = jnp.zeros((16,), jnp.float32)

    @pl.loop(0, CHUNK)
    def _fill_zero(i):
        @pl.loop(0, DEGL // 16)
        def _fz(k):
            buf[i, pl.ds(k * 16, 16)] = zv

    @pl.loop(0, ROWS_PER_TILE // CHUNK)
    def _zero_stripe(z):
        pltpu.sync_copy(
            buf, acc.at[pl.ds(s * ROWS_PER_TILE + z * CHUNK, CHUNK)])

    @pl.loop(0, CHUNK)
    def _fill_ones(i):
        @pl.loop(0, DEGL // 16)
        def _fo(k):
            buf[i, pl.ds(k * 16, 16)] = zv + 1.0

    pltpu.sync_copy(dst_hbm.at[pl.ds(wid * K, K)], dst_v)
    plsc.subcore_barrier()

    @pl.loop(0, K)
    def _scatter(j):
        pltpu.sync_copy(buf, acc.at[dst_v.at[j]], add=True)

    plsc.subcore_barrier()
    pltpu.sync_copy(
        acc.at[pl.ds(s * ROWS_PER_TILE, ROWS_PER_TILE)],
        out_hbm.at[c].at[pl.ds(s * ROWS_PER_TILE, ROWS_PER_TILE)],
    )


def _make_deg_kernel():
    mesh = plsc.VectorSubcoreMesh(core_axis_name="c", subcore_axis_name="s")
    return pl.kernel(
        _deg_body,
        out_type=jax.ShapeDtypeStruct((NC, NP, DEGL), jnp.float32),
        mesh=mesh,
        scratch_types=[
            pltpu.VMEM((K, CHUNK), jnp.int32),
            pltpu.VMEM((CHUNK, DEGL), jnp.float32),
            pltpu.VMEM_SHARED((NP, DEGL), jnp.float32),
            pltpu.SemaphoreType.DMA,
        ],
    )


def _agg_body(g_hbm, src_hbm, dst_hbm, out_hbm, src_v, dst_v, rows,
              acc, sem0, sem1, sem2, sem3):
    c = lax.axis_index("c")
    s = lax.axis_index("s")
    sems = (sem0, sem1, sem2, sem3)

    zv = jnp.zeros((16,), jnp.float32)

    @pl.loop(0, ACH)
    def _fill_zero(i):
        @pl.loop(0, D // 16)
        def _fz(k):
            rows[0, i, pl.ds(k * 16, 16)] = zv

    @pl.loop(0, ROWS_PER_TILE // ACH)
    def _zero_stripe(z):
        pltpu.sync_copy(
            rows.at[0],
            acc.at[pl.ds(s * ROWS_PER_TILE + z * ACH, ACH)],
        )

    plsc.subcore_barrier()

    def _stage(stage_base):
        pltpu.sync_copy(src_hbm.at[pl.ds(stage_base, ASG)], src_v)
        pltpu.sync_copy(dst_hbm.at[pl.ds(stage_base, ASG)], dst_v)

        for b in range(NBUF):
            pltpu.async_copy(g_hbm.at[src_v.at[b]], rows.at[b], sems[b])

        @pl.loop(0, ASG, step=NBUF)
        def _main(j):
            for b in range(NBUF):
                jj = j + b
                sem = sems[b]
                pltpu.make_async_copy(g_hbm.at[src_v.at[jj]], rows.at[b],
                                      sem).wait()
                pltpu.sync_copy(rows.at[b], acc.at[dst_v.at[jj]], add=True)

                @pl.when(jj + NBUF < ASG)
                def _next():
                    pltpu.async_copy(g_hbm.at[src_v.at[jj + NBUF]],
                                     rows.at[b], sem)

    wid = s * NC + c
    for st in range(AK // ASG):
        _stage(wid * AK + st * ASG)

    plsc.subcore_barrier()
    pltpu.sync_copy(
        acc.at[pl.ds(s * ROWS_PER_TILE, ROWS_PER_TILE)],
        out_hbm.at[c].at[pl.ds(s * ROWS_PER_TILE, ROWS_PER_TILE)],
    )


def _make_agg_kernel():
    mesh = plsc.VectorSubcoreMesh(core_axis_name="c", subcore_axis_name="s")
    return pl.kernel(
        _agg_body,
        out_type=jax.ShapeDtypeStruct((NC, NP, D), jnp.float32),
        mesh=mesh,
        scratch_types=[
            pltpu.VMEM((ASG, ACH), jnp.int32),
            pltpu.VMEM((ASG, ACH), jnp.int32),
            pltpu.VMEM((NBUF, ACH, D), jnp.float32),
            pltpu.VMEM_SHARED((NP, D), jnp.float32),
            pltpu.SemaphoreType.DMA,
            pltpu.SemaphoreType.DMA,
            pltpu.SemaphoreType.DMA,
            pltpu.SemaphoreType.DMA,
        ],
    )


def _tc_mm_body(x_ref, w_ref, h_ref):
    h_ref[...] = jnp.dot(x_ref[...], w_ref[...],
                         preferred_element_type=jnp.float32)


def _tc_mm(x_p, W):
    return pl.pallas_call(
        _tc_mm_body,
        grid=(GRID,),
        in_specs=[
            pl.BlockSpec((RB, D), lambda i: (i, 0)),
            pl.BlockSpec((D, H), lambda i: (0, 0)),
        ],
        out_specs=pl.BlockSpec((RB, H), lambda i: (i, 0)),
        out_shape=jax.ShapeDtypeStruct((NP, H), jnp.float32),
    )(x_p, W)


def _tc_a_body(degp_ref, h_ref, g1_ref, dm_ref):
    i = pl.program_id(0)
    d = degp_ref[0, :, 0:1] + degp_ref[1, :, 0:1] + 1.0
    d = jnp.broadcast_to(d, (RB, H))
    rows = i * RB + lax.broadcasted_iota(jnp.int32, (RB, H), 0)
    dm = jnp.where(rows < N, lax.rsqrt(d), 0.0)
    g1_ref[...] = dm * h_ref[...]
    dm_ref[...] = dm


def _tc_a(degp, h1raw):
    return pl.pallas_call(
        _tc_a_body,
        grid=(GRID,),
        in_specs=[
            pl.BlockSpec((NC, RB, DEGL), lambda i: (0, i, 0)),
            pl.BlockSpec((RB, H), lambda i: (i, 0)),
        ],
        out_specs=[
            pl.BlockSpec((RB, H), lambda i: (i, 0)),
            pl.BlockSpec((RB, H), lambda i: (i, 0)),
        ],
        out_shape=[
            jax.ShapeDtypeStruct((NP, H), jnp.float32),
            jax.ShapeDtypeStruct((NP, H), jnp.float32),
        ],
    )(degp, h1raw)


def _tc_b_body(aggp_ref, g1_ref, dm_ref, w2_ref, b1_ref, g2_ref):
    dm = dm_ref[...]
    agg = aggp_ref[0] + aggp_ref[1] + g1_ref[...]
    h1 = jnp.maximum(dm * agg + b1_ref[...], 0.0)
    h2 = jnp.dot(h1, w2_ref[...], preferred_element_type=jnp.float32)
    g2_ref[...] = dm * h2


def _tc_b(aggp, g1, dm, W2, b1_2d):
    return pl.pallas_call(
        _tc_b_body,
        grid=(GRID,),
        in_specs=[
            pl.BlockSpec((NC, RB, H), lambda i: (0, i, 0)),
            pl.BlockSpec((RB, H), lambda i: (i, 0)),
            pl.BlockSpec((RB, H), lambda i: (i, 0)),
            pl.BlockSpec((H, H), lambda i: (0, 0)),
            pl.BlockSpec((1, H), lambda i: (0, 0)),
        ],
        out_specs=pl.BlockSpec((RB, H), lambda i: (i, 0)),
        out_shape=jax.ShapeDtypeStruct((NP, H), jnp.float32),
    )(aggp, g1, dm, W2, b1_2d)


def _tc_c_body(aggp_ref, g2_ref, dm_ref, b2_ref, batch_ref, linw_ref,
               linb_ref, logits_ref, emb_ref, sums, counts):
    i = pl.program_id(0)
    dm = dm_ref[...]
    agg = aggp_ref[0] + aggp_ref[1] + g2_ref[...]
    h2 = jnp.maximum(dm * agg + b2_ref[...], 0.0)

    bvec = batch_ref[0]
    oh_t = (jnp.broadcast_to(bvec, (G, RB))
            == lax.broadcasted_iota(jnp.int32, (G, RB), 0)).astype(jnp.float32)

    @pl.when(i == 0)
    def _init():
        sums[...] = jnp.zeros((G, H), jnp.float32)
        counts[...] = jnp.zeros((G, 1), jnp.float32)

    sums[...] += jnp.dot(oh_t, h2, preferred_element_type=jnp.float32)
    counts[...] += jnp.sum(oh_t, axis=1, keepdims=True)

    @pl.when(i == GRID - 1)
    def _fin():
        cnt = jnp.broadcast_to(jnp.maximum(counts[...], 1.0), (G, H))
        emb = sums[...] / cnt
        emb_ref[...] = emb
        logits_ref[...] = (
            jnp.dot(emb, linw_ref[...], preferred_element_type=jnp.float32)
            + linb_ref[...]
        )


def _tc_c(aggp, g2, dm, b2_2d, batch3d, lin_W, lin_b_2d):
    return pl.pallas_call(
        _tc_c_body,
        grid=(GRID,),
        in_specs=[
            pl.BlockSpec((NC, RB, H), lambda i: (0, i, 0)),
            pl.BlockSpec((RB, H), lambda i: (i, 0)),
            pl.BlockSpec((RB, H), lambda i: (i, 0)),
            pl.BlockSpec((1, H), lambda i: (0, 0)),
            pl.BlockSpec((1, 1, RB), lambda i: (i, 0, 0)),
            pl.BlockSpec((H, OUT), lambda i: (0, 0)),
            pl.BlockSpec((1, OUT), lambda i: (0, 0)),
        ],
        out_specs=[
            pl.BlockSpec((G, OUT), lambda i: (0, 0)),
            pl.BlockSpec((G, H), lambda i: (0, 0)),
        ],
        out_shape=[
            jax.ShapeDtypeStruct((G, OUT), jnp.float32),
            jax.ShapeDtypeStruct((G, H), jnp.float32),
        ],
        scratch_shapes=[
            pltpu.VMEM((G, H), jnp.float32),
            pltpu.VMEM((G, 1), jnp.float32),
        ],
    )(aggp, g2, dm, b2_2d, batch3d, lin_W, lin_b_2d)


@jax.jit
def kernel(x, edge_index, batch, W1, b1, W2, b2, lin_W, lin_b):
    src = edge_index[0]
    dst = edge_index[1]
    pad_i = jnp.arange(EPAD - E, dtype=jnp.int32)
    src_flat = jnp.concatenate([src, pad_i % N])
    dst_flat = jnp.concatenate([dst, N + pad_i % (NP - N)])
    srcs = src_flat.reshape(TOTC, CHUNK)
    dsts = dst_flat.reshape(TOTC, CHUNK)
    srcs_a = src_flat.reshape(ATOTC, ACH)
    dsts_a = dst_flat.reshape(ATOTC, ACH)

    x_p = jnp.pad(x, ((0, NP - N), (0, 0)))
    batch3d = jnp.pad(batch, (0, NP - N), constant_values=G).reshape(
        GRID, 1, RB)

    deg_k = _make_deg_kernel()
    agg_k = _make_agg_kernel()

    degp = deg_k(dsts)
    h1raw = _tc_mm(x_p, W1)
    g1, dm = _tc_a(degp, h1raw)
    agg1 = agg_k(g1, srcs_a, dsts_a)
    g2 = _tc_b(agg1, g1, dm, W2, b1.reshape(1, H))
    agg2 = agg_k(g2, srcs_a, dsts_a)
    logits, embedding = _tc_c(agg2, g2, dm, b2.reshape(1, H), batch3d,
                              lin_W, lin_b.reshape(1, OUT))
    return (logits, embedding)
